# Initial kernel scaffold; baseline (speedup 1.0000x reference)
import jax
import jax.numpy as jnp
from jax.experimental import pallas as pl


def kernel(rois, roi_scores, roi_labels, gt_tracklets, gt_bbs_mask):
    raise NotImplementedError("write your pallas kernel here")


# speedup vs baseline: 1.7384x; 1.7384x over previous
import jax
import jax.numpy as jnp
from jax import lax
from jax.experimental import pallas as pl
from jax.experimental.pallas import tpu as pltpu

_ROI_PER_IMAGE = 128
_FG_RATIO = 0.5
_REG_FG_THRESH = 0.55
_CLS_FG_THRESH = 0.75
_CLS_BG_THRESH = 0.25
_CLS_BG_THRESH_LO = 0.1
_HARD_BG_RATIO = 0.8

_BLK = 2048
_NPAD = 20480
_BIG = 2 ** 30


def _match_compact_kernel(rois_ref, gt_ref, assign_ref, fg_ref, easy_ref,
                          hard_ref, off_ref):
    nb = pl.program_id(1)
    n_roi = 20000

    r = rois_ref[0]
    g = gt_ref[0]

    acx, acy, acz = r[0:1, :], r[1:2, :], r[2:3, :]
    adx, ady, adz = r[3:4, :], r[4:5, :], r[5:6, :]
    bcx, bcy, bcz = g[:, 0:1], g[:, 1:2], g[:, 2:3]
    bdx, bdy, bdz = g[:, 3:4], g[:, 4:5], g[:, 5:6]
    bvalid = g[:, 7:8] > 0.5

    def overlap(ac, ad, bc, bd):
        amin = ac - ad / 2.0
        amax = ac + ad / 2.0
        bmin = bc - bd / 2.0
        bmax = bc + bd / 2.0
        return jnp.maximum(jnp.minimum(amax, bmax) - jnp.maximum(amin, bmin),
                           0.0)

    inter = (overlap(acx, adx, bcx, bdx) * overlap(acy, ady, bcy, bdy)
             * overlap(acz, adz, bcz, bdz))
    va = (adx * ady) * adz
    vb = (bdx * bdy) * bdz
    iou = inter / jnp.maximum(va + vb - inter, 1e-7)
    iou = jnp.where(bvalid, iou, -1.0)

    mo = jnp.max(iou, axis=0, keepdims=True)
    jid = lax.broadcasted_iota(jnp.int32, (128, _BLK), 0)
    assign = jnp.min(jnp.where(iou == mo, jid, _BIG), axis=0, keepdims=True)
    assign_ref[0] = assign

    gidx = nb * _BLK + lax.broadcasted_iota(jnp.int32, (1, _BLK), 1)
    validn = gidx < n_roi
    fg_m = (mo >= _REG_FG_THRESH) & validn
    easy_m = (mo < _CLS_BG_THRESH_LO) & validn
    hard_m = (mo < _REG_FG_THRESH) & (mo >= _CLS_BG_THRESH_LO) & validn

    @pl.when(nb == 0)
    def _init():
        off_ref[0] = 0
        off_ref[1] = 0
        off_ref[2] = 0
        fg_ref[0] = jnp.full((128, 1), _BIG, jnp.int32)
        easy_ref[0] = jnp.full((128, 1), _BIG, jnp.int32)
        hard_ref[0] = jnp.full((128, 1), _BIG, jnp.int32)

    def prefix_sum(x):
        d = 1
        while d < _BLK:
            shifted = jnp.concatenate(
                [jnp.zeros((1, d), x.dtype), x[:, :_BLK - d]], axis=1)
            x = x + shifted
            d *= 2
        return x

    j2 = lax.broadcasted_iota(jnp.int32, (128, _BLK), 0)
    gidx_b = jnp.broadcast_to(gidx, (128, _BLK))
    for c, (m, out_ref) in enumerate(((fg_m, fg_ref), (easy_m, easy_ref),
                                      (hard_m, hard_ref))):
        mi = m.astype(jnp.int32)
        cum = prefix_sum(mi)
        rank = off_ref[c] + cum - 1
        eq = m & (rank == j2)
        cand = jnp.where(eq, gidx_b, _BIG)
        colmin = jnp.min(cand, axis=1, keepdims=True)
        out_ref[0] = jnp.minimum(out_ref[0], colmin)
        off_ref[c] = off_ref[c] + jnp.sum(mi)


def _run_match(rois, gt_tracklets):
    B, N, _ = rois.shape
    n_gt = gt_tracklets.shape[1]

    rois_t = jnp.swapaxes(rois[..., :7], 1, 2)
    rois_t = jnp.pad(rois_t, ((0, 0), (0, 1), (0, _NPAD - N)))

    rowsum = gt_tracklets.sum(-1)
    gidx = jnp.arange(n_gt)
    k = jnp.max(jnp.where(rowsum != 0, gidx[None, :], 0), axis=1)
    col = jnp.arange(128)
    valid = (col[None, :] <= k[:, None]) & (col[None, :] < n_gt)

    gt_pad = jnp.zeros((B, 128, 8), jnp.float32)
    gt_pad = gt_pad.at[:, :n_gt, :7].set(gt_tracklets[..., :7])
    gt_pad = gt_pad.at[:, :, 7].set(valid.astype(jnp.float32))

    nb_blocks = _NPAD // _BLK
    grid = (B, nb_blocks)
    assign, fg, easy, hard = pl.pallas_call(
        _match_compact_kernel,
        grid=grid,
        in_specs=[
            pl.BlockSpec((1, 8, _BLK), lambda b, nb: (b, 0, nb)),
            pl.BlockSpec((1, 128, 8), lambda b, nb: (b, 0, 0)),
        ],
        out_specs=[
            pl.BlockSpec((1, 1, _BLK), lambda b, nb: (b, 0, nb)),
            pl.BlockSpec((1, 128, 1), lambda b, nb: (b, 0, 0)),
            pl.BlockSpec((1, 128, 1), lambda b, nb: (b, 0, 0)),
            pl.BlockSpec((1, 128, 1), lambda b, nb: (b, 0, 0)),
        ],
        out_shape=[
            jax.ShapeDtypeStruct((B, 1, _NPAD), jnp.int32),
            jax.ShapeDtypeStruct((B, 128, 1), jnp.int32),
            jax.ShapeDtypeStruct((B, 128, 1), jnp.int32),
            jax.ShapeDtypeStruct((B, 128, 1), jnp.int32),
        ],
        scratch_shapes=[pltpu.SMEM((4,), jnp.int32)],
    )(rois_t, gt_pad)
    return assign[:, 0, :], fg[..., 0], easy[..., 0], hard[..., 0]


def _select(fg128, easy128, hard128):
    nf = jnp.sum(fg128 < _BIG).astype(jnp.int32)
    ne = jnp.sum(easy128 < _BIG).astype(jnp.int32)
    nh_cnt = jnp.sum(hard128 < _BIG).astype(jnp.int32)
    fg128 = jnp.where(fg128 < _BIG, fg128, 0)
    easy128 = jnp.where(easy128 < _BIG, easy128, 0)
    hard128 = jnp.where(hard128 < _BIG, hard128, 0)

    R = _ROI_PER_IMAGE
    fg_per = int(round(_FG_RATIO * R))
    p = jnp.arange(R)

    def pick(arr, cnt, pos):
        return arr[pos % jnp.maximum(cnt, 1)]

    def sample_bg(pos, n):
        nh = jnp.floor(n * _HARD_BG_RATIO).astype(jnp.int32)
        both_val = jnp.where(pos < nh,
                             pick(hard128, nh_cnt, pos),
                             pick(easy128, ne, pos - nh))
        hard_only = pick(hard128, nh_cnt, pos)
        easy_only = pick(easy128, ne, pos)
        return jnp.where((nh_cnt > 0) & (ne > 0), both_val,
                         jnp.where(nh_cnt > 0, hard_only, easy_only))

    nb = ne + nh_cnt
    nfg = jnp.minimum(fg_per, nf)
    mixed = jnp.where(p < nfg, fg128[p], sample_bg(p - nfg, R - nfg))
    fg_only = pick(fg128, nf, p)
    bg_only = sample_bg(p, R)
    inds = jnp.where((nf > 0) & (nb > 0), mixed,
                     jnp.where(nf > 0, fg_only,
                               jnp.where(nb > 0, bg_only, 0)))
    return inds


def _iou3d_elem(a, b):
    a_min = a[..., 0:3] - a[..., 3:6] / 2.0
    a_max = a[..., 0:3] + a[..., 3:6] / 2.0
    b_min = b[..., 0:3] - b[..., 3:6] / 2.0
    b_max = b[..., 0:3] + b[..., 3:6] / 2.0
    inter = jnp.maximum(jnp.minimum(a_max, b_max) - jnp.maximum(a_min, b_min),
                        0.0).prod(-1)
    va = a[..., 3:6].prod(-1)
    vb = b[..., 3:6].prod(-1)
    return inter / jnp.maximum(va + vb - inter, 1e-7)


def kernel(rois, roi_scores, roi_labels, gt_tracklets, gt_bbs_mask):
    assign_full, fg, easy, hard = _run_match(rois, gt_tracklets)
    sampled_inds = jax.vmap(_select)(fg, easy, hard)

    B = rois.shape[0]
    F = gt_tracklets.shape[-1] // 7
    bidx = jnp.arange(B)[:, None]
    gt_assign = assign_full[bidx, sampled_inds]

    batch_rois = rois[bidx, sampled_inds]
    batch_gt_of_rois = gt_tracklets[bidx, gt_assign]
    batch_roi_scores = roi_scores[bidx, sampled_inds]
    batch_roi_labels = roi_labels[bidx, sampled_inds]
    batch_gt_bbs_mask = gt_bbs_mask[bidx, gt_assign]

    ious = [_iou3d_elem(batch_rois[..., 0:7], batch_gt_of_rois[..., 0:7])]
    for i in range(1, F):
        ious.append(_iou3d_elem(batch_rois[..., i * 7:i * 7 + 7],
                                batch_gt_of_rois[..., i * 7:i * 7 + 7]))
    batch_all_roi_ious = jnp.stack(ious, -1)
    tracks_mean_ious = batch_all_roi_ious.sum(-1) / (batch_gt_bbs_mask.sum(-1)
                                                     + 1e-5)
    batch_roi_ious = batch_all_roi_ious[..., 0]
    reg_valid_mask = (batch_roi_ious > _REG_FG_THRESH).astype(jnp.int32)
    fg_mask = batch_roi_ious > _CLS_FG_THRESH
    bg_mask = batch_roi_ious < _CLS_BG_THRESH
    interval_mask = (~fg_mask) & (~bg_mask)
    cls = jnp.where(fg_mask, 1.0, 0.0)
    cls = jnp.where(interval_mask,
                    (batch_roi_ious - _CLS_BG_THRESH)
                    / (_CLS_FG_THRESH - _CLS_BG_THRESH), cls)
    return (batch_rois, batch_gt_of_rois, batch_roi_ious, batch_roi_scores,
            batch_roi_labels, reg_valid_mask, batch_gt_bbs_mask, cls,
            tracks_mean_ious)


# speedup vs baseline: 1.8067x; 1.0393x over previous
import jax
import jax.numpy as jnp
from jax import lax
from jax.experimental import pallas as pl
from jax.experimental.pallas import tpu as pltpu

_ROI_PER_IMAGE = 128
_FG_RATIO = 0.5
_REG_FG_THRESH = 0.55
_CLS_FG_THRESH = 0.75
_CLS_BG_THRESH = 0.25
_CLS_BG_THRESH_LO = 0.1
_HARD_BG_RATIO = 0.8

_BLK = 4096
_NPAD = 20480
_BIG = 2 ** 30


def _match_compact_kernel(rois_ref, gt_ref, assign_ref, fg_ref, easy_ref,
                          hard_ref, off_ref):
    nb = pl.program_id(1)
    n_roi = 20000

    r = rois_ref[0]
    g = gt_ref[0]

    acx, acy, acz = r[0:1, :], r[1:2, :], r[2:3, :]
    adx, ady, adz = r[3:4, :], r[4:5, :], r[5:6, :]
    bcx, bcy, bcz = g[:, 0:1], g[:, 1:2], g[:, 2:3]
    bdx, bdy, bdz = g[:, 3:4], g[:, 4:5], g[:, 5:6]
    bvalid = g[:, 7:8] > 0.5

    def overlap(ac, ad, bc, bd):
        amin = ac - ad / 2.0
        amax = ac + ad / 2.0
        bmin = bc - bd / 2.0
        bmax = bc + bd / 2.0
        return jnp.maximum(jnp.minimum(amax, bmax) - jnp.maximum(amin, bmin),
                           0.0)

    inter = (overlap(acx, adx, bcx, bdx) * overlap(acy, ady, bcy, bdy)
             * overlap(acz, adz, bcz, bdz))
    va = (adx * ady) * adz
    vb = (bdx * bdy) * bdz
    iou = inter / jnp.maximum(va + vb - inter, 1e-7)
    iou = jnp.where(bvalid, iou, -1.0)

    mo = jnp.max(iou, axis=0, keepdims=True)
    jid = lax.broadcasted_iota(jnp.int32, (128, _BLK), 0)
    assign = jnp.min(jnp.where(iou == mo, jid, _BIG), axis=0, keepdims=True)
    assign_ref[0] = assign

    gidx = nb * _BLK + lax.broadcasted_iota(jnp.int32, (1, _BLK), 1)
    validn = gidx < n_roi
    fg_m = (mo >= _REG_FG_THRESH) & validn
    easy_m = (mo < _CLS_BG_THRESH_LO) & validn
    hard_m = (mo < _REG_FG_THRESH) & (mo >= _CLS_BG_THRESH_LO) & validn

    @pl.when(nb == 0)
    def _init():
        off_ref[0] = 0
        off_ref[1] = 0
        off_ref[2] = 0
        fg_ref[0] = jnp.full((128, 1), _BIG, jnp.int32)
        easy_ref[0] = jnp.full((128, 1), _BIG, jnp.int32)
        hard_ref[0] = jnp.full((128, 1), _BIG, jnp.int32)

    def prefix_sum(x):
        d = 1
        while d < _BLK:
            shifted = jnp.concatenate(
                [jnp.zeros((1, d), x.dtype), x[:, :_BLK - d]], axis=1)
            x = x + shifted
            d *= 2
        return x

    j2 = lax.broadcasted_iota(jnp.int32, (128, _BLK), 0)
    gidx_b = jnp.broadcast_to(gidx, (128, _BLK))
    for c, (m, out_ref) in enumerate(((fg_m, fg_ref), (easy_m, easy_ref),
                                      (hard_m, hard_ref))):
        mi = m.astype(jnp.int32)
        cum = prefix_sum(mi)
        rank = off_ref[c] + cum - 1
        eq = m & (rank == j2)
        cand = jnp.where(eq, gidx_b, _BIG)
        colmin = jnp.min(cand, axis=1, keepdims=True)
        out_ref[0] = jnp.minimum(out_ref[0], colmin)
        off_ref[c] = off_ref[c] + jnp.sum(mi)


def _run_match(rois, gt_tracklets):
    B, N, _ = rois.shape
    n_gt = gt_tracklets.shape[1]

    rois_t = jnp.swapaxes(rois[..., :7], 1, 2)
    rois_t = jnp.pad(rois_t, ((0, 0), (0, 1), (0, _NPAD - N)))

    rowsum = gt_tracklets.sum(-1)
    gidx = jnp.arange(n_gt)
    k = jnp.max(jnp.where(rowsum != 0, gidx[None, :], 0), axis=1)
    col = jnp.arange(128)
    valid = (col[None, :] <= k[:, None]) & (col[None, :] < n_gt)

    gt_pad = jnp.zeros((B, 128, 8), jnp.float32)
    gt_pad = gt_pad.at[:, :n_gt, :7].set(gt_tracklets[..., :7])
    gt_pad = gt_pad.at[:, :, 7].set(valid.astype(jnp.float32))

    nb_blocks = _NPAD // _BLK
    grid = (B, nb_blocks)
    assign, fg, easy, hard = pl.pallas_call(
        _match_compact_kernel,
        grid=grid,
        in_specs=[
            pl.BlockSpec((1, 8, _BLK), lambda b, nb: (b, 0, nb)),
            pl.BlockSpec((1, 128, 8), lambda b, nb: (b, 0, 0)),
        ],
        out_specs=[
            pl.BlockSpec((1, 1, _BLK), lambda b, nb: (b, 0, nb)),
            pl.BlockSpec((1, 128, 1), lambda b, nb: (b, 0, 0)),
            pl.BlockSpec((1, 128, 1), lambda b, nb: (b, 0, 0)),
            pl.BlockSpec((1, 128, 1), lambda b, nb: (b, 0, 0)),
        ],
        out_shape=[
            jax.ShapeDtypeStruct((B, 1, _NPAD), jnp.int32),
            jax.ShapeDtypeStruct((B, 128, 1), jnp.int32),
            jax.ShapeDtypeStruct((B, 128, 1), jnp.int32),
            jax.ShapeDtypeStruct((B, 128, 1), jnp.int32),
        ],
        scratch_shapes=[pltpu.SMEM((4,), jnp.int32)],
    )(rois_t, gt_pad)
    return assign[:, 0, :], fg[..., 0], easy[..., 0], hard[..., 0]


def _select(fg128, easy128, hard128):
    nf = jnp.sum(fg128 < _BIG).astype(jnp.int32)
    ne = jnp.sum(easy128 < _BIG).astype(jnp.int32)
    nh_cnt = jnp.sum(hard128 < _BIG).astype(jnp.int32)
    fg128 = jnp.where(fg128 < _BIG, fg128, 0)
    easy128 = jnp.where(easy128 < _BIG, easy128, 0)
    hard128 = jnp.where(hard128 < _BIG, hard128, 0)

    R = _ROI_PER_IMAGE
    fg_per = int(round(_FG_RATIO * R))
    p = jnp.arange(R)

    def pick(arr, cnt, pos):
        return arr[pos % jnp.maximum(cnt, 1)]

    def sample_bg(pos, n):
        nh = jnp.floor(n * _HARD_BG_RATIO).astype(jnp.int32)
        both_val = jnp.where(pos < nh,
                             pick(hard128, nh_cnt, pos),
                             pick(easy128, ne, pos - nh))
        hard_only = pick(hard128, nh_cnt, pos)
        easy_only = pick(easy128, ne, pos)
        return jnp.where((nh_cnt > 0) & (ne > 0), both_val,
                         jnp.where(nh_cnt > 0, hard_only, easy_only))

    nb = ne + nh_cnt
    nfg = jnp.minimum(fg_per, nf)
    mixed = jnp.where(p < nfg, fg128[p], sample_bg(p - nfg, R - nfg))
    fg_only = pick(fg128, nf, p)
    bg_only = sample_bg(p, R)
    inds = jnp.where((nf > 0) & (nb > 0), mixed,
                     jnp.where(nf > 0, fg_only,
                               jnp.where(nb > 0, bg_only, 0)))
    return inds


def _iou3d_elem(a, b):
    a_min = a[..., 0:3] - a[..., 3:6] / 2.0
    a_max = a[..., 0:3] + a[..., 3:6] / 2.0
    b_min = b[..., 0:3] - b[..., 3:6] / 2.0
    b_max = b[..., 0:3] + b[..., 3:6] / 2.0
    inter = jnp.maximum(jnp.minimum(a_max, b_max) - jnp.maximum(a_min, b_min),
                        0.0).prod(-1)
    va = a[..., 3:6].prod(-1)
    vb = b[..., 3:6].prod(-1)
    return inter / jnp.maximum(va + vb - inter, 1e-7)


def kernel(rois, roi_scores, roi_labels, gt_tracklets, gt_bbs_mask):
    assign_full, fg, easy, hard = _run_match(rois, gt_tracklets)
    sampled_inds = jax.vmap(_select)(fg, easy, hard)

    B = rois.shape[0]
    F = gt_tracklets.shape[-1] // 7
    bidx = jnp.arange(B)[:, None]
    gt_assign = assign_full[bidx, sampled_inds]

    batch_rois = rois[bidx, sampled_inds]
    batch_gt_of_rois = gt_tracklets[bidx, gt_assign]
    batch_roi_scores = roi_scores[bidx, sampled_inds]
    batch_roi_labels = roi_labels[bidx, sampled_inds]
    batch_gt_bbs_mask = gt_bbs_mask[bidx, gt_assign]

    ious = [_iou3d_elem(batch_rois[..., 0:7], batch_gt_of_rois[..., 0:7])]
    for i in range(1, F):
        ious.append(_iou3d_elem(batch_rois[..., i * 7:i * 7 + 7],
                                batch_gt_of_rois[..., i * 7:i * 7 + 7]))
    batch_all_roi_ious = jnp.stack(ious, -1)
    tracks_mean_ious = batch_all_roi_ious.sum(-1) / (batch_gt_bbs_mask.sum(-1)
                                                     + 1e-5)
    batch_roi_ious = batch_all_roi_ious[..., 0]
    reg_valid_mask = (batch_roi_ious > _REG_FG_THRESH).astype(jnp.int32)
    fg_mask = batch_roi_ious > _CLS_FG_THRESH
    bg_mask = batch_roi_ious < _CLS_BG_THRESH
    interval_mask = (~fg_mask) & (~bg_mask)
    cls = jnp.where(fg_mask, 1.0, 0.0)
    cls = jnp.where(interval_mask,
                    (batch_roi_ious - _CLS_BG_THRESH)
                    / (_CLS_FG_THRESH - _CLS_BG_THRESH), cls)
    return (batch_rois, batch_gt_of_rois, batch_roi_ious, batch_roi_scores,
            batch_roi_labels, reg_valid_mask, batch_gt_bbs_mask, cls,
            tracks_mean_ious)


# speedup vs baseline: 1.9488x; 1.0787x over previous
import jax
import jax.numpy as jnp
from jax import lax
from jax.experimental import pallas as pl
from jax.experimental.pallas import tpu as pltpu

_ROI_PER_IMAGE = 128
_FG_RATIO = 0.5
_REG_FG_THRESH = 0.55
_CLS_FG_THRESH = 0.75
_CLS_BG_THRESH = 0.25
_CLS_BG_THRESH_LO = 0.1
_HARD_BG_RATIO = 0.8

_BLK = 4096
_NPAD = 20480
_BIG = 2 ** 30


def _match_compact_kernel(rois_ref, gt_ref, assign_ref, fg_ref, easy_ref,
                          hard_ref, off_ref):
    nb = pl.program_id(1)
    n_roi = 20000

    r = rois_ref[0]
    g = gt_ref[0]
    n_gt_pad = g.shape[0]

    acx, acy, acz = r[0:1, :], r[1:2, :], r[2:3, :]
    adx, ady, adz = r[3:4, :], r[4:5, :], r[5:6, :]
    bcx, bcy, bcz = g[:, 0:1], g[:, 1:2], g[:, 2:3]
    bdx, bdy, bdz = g[:, 3:4], g[:, 4:5], g[:, 5:6]
    bvalid = g[:, 7:8] > 0.5

    def overlap(ac, ad, bc, bd):
        amin = ac - ad / 2.0
        amax = ac + ad / 2.0
        bmin = bc - bd / 2.0
        bmax = bc + bd / 2.0
        return jnp.maximum(jnp.minimum(amax, bmax) - jnp.maximum(amin, bmin),
                           0.0)

    inter = (overlap(acx, adx, bcx, bdx) * overlap(acy, ady, bcy, bdy)
             * overlap(acz, adz, bcz, bdz))
    va = (adx * ady) * adz
    vb = (bdx * bdy) * bdz
    iou = inter / jnp.maximum(va + vb - inter, 1e-7)
    iou = jnp.where(bvalid, iou, -1.0)

    mo = jnp.max(iou, axis=0, keepdims=True)
    jid = lax.broadcasted_iota(jnp.int32, (n_gt_pad, _BLK), 0)
    assign = jnp.min(jnp.where(iou == mo, jid, _BIG), axis=0, keepdims=True)
    assign_ref[0] = assign

    gidx = nb * _BLK + lax.broadcasted_iota(jnp.int32, (1, _BLK), 1)
    validn = gidx < n_roi
    fg_m = (mo >= _REG_FG_THRESH) & validn
    easy_m = (mo < _CLS_BG_THRESH_LO) & validn
    hard_m = (mo < _REG_FG_THRESH) & (mo >= _CLS_BG_THRESH_LO) & validn

    @pl.when(nb == 0)
    def _init():
        off_ref[0] = 0
        off_ref[1] = 0
        off_ref[2] = 0
        fg_ref[0] = jnp.full((128, 1), _BIG, jnp.int32)
        easy_ref[0] = jnp.full((128, 1), _BIG, jnp.int32)
        hard_ref[0] = jnp.full((128, 1), _BIG, jnp.int32)

    def prefix_sum(x):
        d = 1
        while d < _BLK:
            shifted = jnp.concatenate(
                [jnp.zeros((1, d), x.dtype), x[:, :_BLK - d]], axis=1)
            x = x + shifted
            d *= 2
        return x

    j2 = lax.broadcasted_iota(jnp.int32, (128, _BLK), 0)
    gidx_b = jnp.broadcast_to(gidx, (128, _BLK))
    for c, (m, out_ref) in enumerate(((fg_m, fg_ref), (easy_m, easy_ref),
                                      (hard_m, hard_ref))):
        mi = m.astype(jnp.int32)

        @pl.when(off_ref[c] < 128)
        def _compact(c=c, m=m, mi=mi, out_ref=out_ref):
            cum = prefix_sum(mi)
            rank = off_ref[c] + cum - 1
            eq = m & (rank == j2)
            cand = jnp.where(eq, gidx_b, _BIG)
            colmin = jnp.min(cand, axis=1, keepdims=True)
            out_ref[0] = jnp.minimum(out_ref[0], colmin)

        off_ref[c] = off_ref[c] + jnp.sum(mi)


def _run_match(rois, gt_tracklets):
    B, N, _ = rois.shape
    n_gt = gt_tracklets.shape[1]

    rois_t = jnp.swapaxes(rois[..., :7], 1, 2)
    rois_t = jnp.pad(rois_t, ((0, 0), (0, 1), (0, _NPAD - N)))

    rowsum = gt_tracklets.sum(-1)
    gidx = jnp.arange(n_gt)
    k = jnp.max(jnp.where(rowsum != 0, gidx[None, :], 0), axis=1)
    col = jnp.arange(104)
    valid = (col[None, :] <= k[:, None]) & (col[None, :] < n_gt)

    gt_pad = jnp.zeros((B, 104, 8), jnp.float32)
    gt_pad = gt_pad.at[:, :n_gt, :7].set(gt_tracklets[..., :7])
    gt_pad = gt_pad.at[:, :, 7].set(valid.astype(jnp.float32))

    nb_blocks = _NPAD // _BLK
    grid = (B, nb_blocks)
    assign, fg, easy, hard = pl.pallas_call(
        _match_compact_kernel,
        grid=grid,
        in_specs=[
            pl.BlockSpec((1, 8, _BLK), lambda b, nb: (b, 0, nb)),
            pl.BlockSpec((1, 104, 8), lambda b, nb: (b, 0, 0)),
        ],
        out_specs=[
            pl.BlockSpec((1, 1, _BLK), lambda b, nb: (b, 0, nb)),
            pl.BlockSpec((1, 128, 1), lambda b, nb: (b, 0, 0)),
            pl.BlockSpec((1, 128, 1), lambda b, nb: (b, 0, 0)),
            pl.BlockSpec((1, 128, 1), lambda b, nb: (b, 0, 0)),
        ],
        out_shape=[
            jax.ShapeDtypeStruct((B, 1, _NPAD), jnp.int32),
            jax.ShapeDtypeStruct((B, 128, 1), jnp.int32),
            jax.ShapeDtypeStruct((B, 128, 1), jnp.int32),
            jax.ShapeDtypeStruct((B, 128, 1), jnp.int32),
        ],
        scratch_shapes=[pltpu.SMEM((4,), jnp.int32)],
    )(rois_t, gt_pad)
    return assign[:, 0, :], fg[..., 0], easy[..., 0], hard[..., 0]


def _select(fg128, easy128, hard128):
    nf = jnp.sum(fg128 < _BIG).astype(jnp.int32)
    ne = jnp.sum(easy128 < _BIG).astype(jnp.int32)
    nh_cnt = jnp.sum(hard128 < _BIG).astype(jnp.int32)
    fg128 = jnp.where(fg128 < _BIG, fg128, 0)
    easy128 = jnp.where(easy128 < _BIG, easy128, 0)
    hard128 = jnp.where(hard128 < _BIG, hard128, 0)

    R = _ROI_PER_IMAGE
    fg_per = int(round(_FG_RATIO * R))
    p = jnp.arange(R)

    def pick(arr, cnt, pos):
        return arr[pos % jnp.maximum(cnt, 1)]

    def sample_bg(pos, n):
        nh = jnp.floor(n * _HARD_BG_RATIO).astype(jnp.int32)
        both_val = jnp.where(pos < nh,
                             pick(hard128, nh_cnt, pos),
                             pick(easy128, ne, pos - nh))
        hard_only = pick(hard128, nh_cnt, pos)
        easy_only = pick(easy128, ne, pos)
        return jnp.where((nh_cnt > 0) & (ne > 0), both_val,
                         jnp.where(nh_cnt > 0, hard_only, easy_only))

    nb = ne + nh_cnt
    nfg = jnp.minimum(fg_per, nf)
    mixed = jnp.where(p < nfg, fg128[p], sample_bg(p - nfg, R - nfg))
    fg_only = pick(fg128, nf, p)
    bg_only = sample_bg(p, R)
    inds = jnp.where((nf > 0) & (nb > 0), mixed,
                     jnp.where(nf > 0, fg_only,
                               jnp.where(nb > 0, bg_only, 0)))
    return inds


def _iou3d_elem(a, b):
    a_min = a[..., 0:3] - a[..., 3:6] / 2.0
    a_max = a[..., 0:3] + a[..., 3:6] / 2.0
    b_min = b[..., 0:3] - b[..., 3:6] / 2.0
    b_max = b[..., 0:3] + b[..., 3:6] / 2.0
    inter = jnp.maximum(jnp.minimum(a_max, b_max) - jnp.maximum(a_min, b_min),
                        0.0).prod(-1)
    va = a[..., 3:6].prod(-1)
    vb = b[..., 3:6].prod(-1)
    return inter / jnp.maximum(va + vb - inter, 1e-7)


def kernel(rois, roi_scores, roi_labels, gt_tracklets, gt_bbs_mask):
    assign_full, fg, easy, hard = _run_match(rois, gt_tracklets)
    sampled_inds = jax.vmap(_select)(fg, easy, hard)

    B = rois.shape[0]
    F = gt_tracklets.shape[-1] // 7
    bidx = jnp.arange(B)[:, None]
    gt_assign = assign_full[bidx, sampled_inds]

    batch_rois = rois[bidx, sampled_inds]
    batch_gt_of_rois = gt_tracklets[bidx, gt_assign]
    batch_roi_scores = roi_scores[bidx, sampled_inds]
    batch_roi_labels = roi_labels[bidx, sampled_inds]
    batch_gt_bbs_mask = gt_bbs_mask[bidx, gt_assign]

    ious = [_iou3d_elem(batch_rois[..., 0:7], batch_gt_of_rois[..., 0:7])]
    for i in range(1, F):
        ious.append(_iou3d_elem(batch_rois[..., i * 7:i * 7 + 7],
                                batch_gt_of_rois[..., i * 7:i * 7 + 7]))
    batch_all_roi_ious = jnp.stack(ious, -1)
    tracks_mean_ious = batch_all_roi_ious.sum(-1) / (batch_gt_bbs_mask.sum(-1)
                                                     + 1e-5)
    batch_roi_ious = batch_all_roi_ious[..., 0]
    reg_valid_mask = (batch_roi_ious > _REG_FG_THRESH).astype(jnp.int32)
    fg_mask = batch_roi_ious > _CLS_FG_THRESH
    bg_mask = batch_roi_ious < _CLS_BG_THRESH
    interval_mask = (~fg_mask) & (~bg_mask)
    cls = jnp.where(fg_mask, 1.0, 0.0)
    cls = jnp.where(interval_mask,
                    (batch_roi_ious - _CLS_BG_THRESH)
                    / (_CLS_FG_THRESH - _CLS_BG_THRESH), cls)
    return (batch_rois, batch_gt_of_rois, batch_roi_ious, batch_roi_scores,
            batch_roi_labels, reg_valid_mask, batch_gt_bbs_mask, cls,
            tracks_mean_ious)
